# Initial kernel scaffold; baseline (speedup 1.0000x reference)
import jax
import jax.numpy as jnp
from jax.experimental import pallas as pl


def kernel(objectness, pred_bbox_deltas, anchors):
    raise NotImplementedError("write your pallas kernel here")


# speedup vs baseline: 22.7844x; 22.7844x over previous
import math
import jax
import jax.numpy as jnp
from jax import lax
from jax.experimental import pallas as pl
from jax.experimental.pallas import tpu as pltpu

_NUM_ANCHORS_PER_LEVEL = [160000, 40000]
_PRE_NMS_TOP_N = 1000
_POST_NMS_TOP_N = 1000
_NMS_THRESH = 0.7
_MIN_SIZE = 0.001
_IMG_H = 800.0
_IMG_W = 800.0
_BBOX_XFORM_CLIP = math.log(1000.0 / 16)
_NCAND = 2000
_NPAD = 2048
_ROWS = 16
_LANES = 128
_LVL_OFFSET = 801.0


def _nms_pipeline_kernel(obj_ref, lvl_ref, deltas_ref, anchors_ref,
                         boxes_ref, keep_ref):
    a = anchors_ref[0]
    d = deltas_ref[0]
    obj = obj_ref[0]
    lvl = lvl_ref[0]

    widths = a[2] - a[0]
    heights = a[3] - a[1]
    ctr_x = a[0] + 0.5 * widths
    ctr_y = a[1] + 0.5 * heights
    dx = d[0]
    dy = d[1]
    dw = jnp.minimum(d[2], _BBOX_XFORM_CLIP)
    dh = jnp.minimum(d[3], _BBOX_XFORM_CLIP)
    pred_ctr_x = dx * widths + ctr_x
    pred_ctr_y = dy * heights + ctr_y
    pred_w = jnp.exp(dw) * widths
    pred_h = jnp.exp(dh) * heights

    x1 = jnp.clip(pred_ctr_x - 0.5 * pred_w, 0.0, _IMG_W)
    y1 = jnp.clip(pred_ctr_y - 0.5 * pred_h, 0.0, _IMG_H)
    x2 = jnp.clip(pred_ctr_x + 0.5 * pred_w, 0.0, _IMG_W)
    y2 = jnp.clip(pred_ctr_y + 0.5 * pred_h, 0.0, _IMG_H)

    ws = x2 - x1
    hs = y2 - y1
    probs = jax.nn.sigmoid(obj)
    valid = (ws >= _MIN_SIZE) & (hs >= _MIN_SIZE) & (probs >= 0.0)

    off = lvl * _LVL_OFFSET
    nx1 = x1 + off
    ny1 = y1 + off
    nx2 = x2 + off
    ny2 = y2 + off
    areas = ws * hs

    flatidx = (lax.broadcasted_iota(jnp.int32, (_ROWS, _LANES), 0) * _LANES
               + lax.broadcasted_iota(jnp.int32, (_ROWS, _LANES), 1))
    keep0 = jnp.where(valid, 1.0, 0.0)

    def body(i, keep):
        onehot = flatidx == i
        ki = jnp.sum(jnp.where(onehot, keep, 0.0))
        bx1 = jnp.sum(jnp.where(onehot, nx1, 0.0))
        by1 = jnp.sum(jnp.where(onehot, ny1, 0.0))
        bx2 = jnp.sum(jnp.where(onehot, nx2, 0.0))
        by2 = jnp.sum(jnp.where(onehot, ny2, 0.0))
        ai = jnp.sum(jnp.where(onehot, areas, 0.0))
        xx1 = jnp.maximum(bx1, nx1)
        yy1 = jnp.maximum(by1, ny1)
        xx2 = jnp.minimum(bx2, nx2)
        yy2 = jnp.minimum(by2, ny2)
        inter = jnp.maximum(xx2 - xx1, 0.0) * jnp.maximum(yy2 - yy1, 0.0)
        iou = inter / (ai + areas - inter + 1e-9)
        sup = (iou > _NMS_THRESH) & (flatidx > i) & (ki > 0.0)
        return jnp.where(sup, 0.0, keep)

    keep = lax.fori_loop(0, _NCAND, body, keep0)

    keep_ref[0] = keep
    boxes_ref[0, 0] = x1
    boxes_ref[0, 1] = y1
    boxes_ref[0, 2] = x2
    boxes_ref[0, 3] = y2


def kernel(objectness, pred_bbox_deltas, anchors):
    objectness = lax.stop_gradient(objectness)
    deltas = lax.stop_gradient(pred_bbox_deltas)
    nimg = objectness.shape[0]

    top_idx = []
    off = 0
    for n in _NUM_ANCHORS_PER_LEVEL:
        k = min(_PRE_NMS_TOP_N, n)
        _, idx = lax.top_k(objectness[:, off:off + n], k)
        top_idx.append(idx + off)
        off += n
    top_idx = jnp.concatenate(top_idx, axis=1)
    bidx = jnp.arange(nimg)[:, None]

    obj = objectness[bidx, top_idx]
    lvl = (top_idx >= _NUM_ANCHORS_PER_LEVEL[0]).astype(jnp.float32)
    dts = deltas[bidx, top_idx]
    anc = anchors[top_idx]

    pad = _NPAD - _NCAND
    obj = jnp.pad(obj, ((0, 0), (0, pad)), constant_values=-1e30)
    lvl = jnp.pad(lvl, ((0, 0), (0, pad)))
    dts = jnp.pad(dts, ((0, 0), (0, pad), (0, 0)))
    anc = jnp.pad(anc, ((0, 0), (0, pad), (0, 0)))

    order = jnp.argsort(-obj, axis=1)
    obj_s = jnp.take_along_axis(obj, order, axis=1)
    lvl_s = jnp.take_along_axis(lvl, order, axis=1)
    dts_s = jnp.take_along_axis(dts, order[..., None], axis=1)
    anc_s = jnp.take_along_axis(anc, order[..., None], axis=1)

    obj_v = obj_s.reshape(nimg, _ROWS, _LANES)
    lvl_v = lvl_s.reshape(nimg, _ROWS, _LANES)
    dts_v = dts_s.transpose(0, 2, 1).reshape(nimg, 4, _ROWS, _LANES)
    anc_v = anc_s.transpose(0, 2, 1).reshape(nimg, 4, _ROWS, _LANES)

    boxes_v, keep_v = pl.pallas_call(
        _nms_pipeline_kernel,
        grid=(nimg,),
        in_specs=[
            pl.BlockSpec((1, _ROWS, _LANES), lambda i: (i, 0, 0)),
            pl.BlockSpec((1, _ROWS, _LANES), lambda i: (i, 0, 0)),
            pl.BlockSpec((1, 4, _ROWS, _LANES), lambda i: (i, 0, 0, 0)),
            pl.BlockSpec((1, 4, _ROWS, _LANES), lambda i: (i, 0, 0, 0)),
        ],
        out_specs=[
            pl.BlockSpec((1, 4, _ROWS, _LANES), lambda i: (i, 0, 0, 0)),
            pl.BlockSpec((1, _ROWS, _LANES), lambda i: (i, 0, 0)),
        ],
        out_shape=[
            jax.ShapeDtypeStruct((nimg, 4, _ROWS, _LANES), jnp.float32),
            jax.ShapeDtypeStruct((nimg, _ROWS, _LANES), jnp.float32),
        ],
    )(obj_v, lvl_v, dts_v, anc_v)

    boxes_s = boxes_v.reshape(nimg, 4, _NPAD).transpose(0, 2, 1)
    keep = keep_v.reshape(nimg, _NPAD) > 0.5

    rank = jnp.cumsum(keep.astype(jnp.int32), axis=1) - 1
    dest = jnp.where(keep & (rank < _POST_NMS_TOP_N), rank, _POST_NMS_TOP_N)
    out = jnp.zeros((nimg, _POST_NMS_TOP_N + 1, 4), jnp.float32)
    out = out.at[bidx, dest].set(boxes_s, mode="drop")
    return out[:, :_POST_NMS_TOP_N]
